# baseline (speedup 1.0000x reference)
import dataclasses
import functools

import jax
import jax.numpy as jnp
from jax import lax
from jax.experimental import pallas as pl
from jax.experimental.pallas import tpu as pltpu
from jax.experimental.pallas import tpu_sc as plsc

POOL = 7
SCALE = 0.0625
B, C, H, W = 2, 256, 38, 38
N = 1000
NEG = jnp.finfo(jnp.float32).min

HW = H * W
NCOMBO = 9
ZERO_ROW = NCOMBO * B * HW

NBINS = N * POOL * POOL
NWORKERS = 32
CB = 32
BINS_PER_W = 1536
BINSP = NWORKERS * BINS_PER_W
NCHUNK = BINS_PER_W // CB


def _shift0(x, s, size):
    return jnp.concatenate([x[s:], jnp.broadcast_to(x[size - 1:], (s,) + x.shape[1:])], axis=0)


def _shift1(x, s, size):
    last = x[:, size - 1:]
    return jnp.concatenate([x[:, s:], jnp.broadcast_to(last, x.shape[:1] + (s,) + x.shape[2:])], axis=1)


def _table_body(fmap_ref, out_ref):
    for bb in range(B):
        f = fmap_ref[bb]
        th = f
        for kh in range(3):
            if kh:
                th = jnp.maximum(th, _shift0(th, 1 << (kh - 1), H))
            tw = th
            for kw in range(3):
                if kw:
                    tw = jnp.maximum(tw, _shift1(tw, 1 << (kw - 1), W))
                out_ref[(kh * 3 + kw) * B + bb] = tw.astype(jnp.bfloat16)
    out_ref[NCOMBO * B] = jnp.zeros((H, W, C), jnp.bfloat16)


def _sc_body(table_hbm, idx_hbm, out_hbm, idx_v, rows_v, out_v, sem0, sem1):
    wid = lax.axis_index("s") * 2 + lax.axis_index("c")
    base = wid * BINS_PER_W

    def start_gather(t, buf, sem):
        pltpu.sync_copy(idx_hbm.at[pl.ds((base + t * CB) * 4, CB * 4)], idx_v.at[buf])
        return pltpu.async_copy(table_hbm.at[idx_v.at[buf]], rows_v.at[buf], sem)

    def compute(buf):
        def bin_body(i):
            for j in range(C // 32):
                sl = pl.ds(j * 16, 16)
                r = [plsc.bitcast(rows_v[buf, 4 * i + k, sl], jnp.bfloat16)
                     for k in range(4)]
                m = jnp.maximum(jnp.maximum(r[0], r[1]), jnp.maximum(r[2], r[3]))
                out_v[i, sl] = plsc.bitcast(m, jnp.int32)
        pl.loop(0, CB)(bin_body)

    start_gather(0, 0, sem0).wait()

    def chunk_body(t):
        @pl.when(t % 2 == 0)
        def _():
            cp = start_gather(t + 1, 1, sem1)
            compute(0)
            cp.wait()

        @pl.when(t % 2 == 1)
        def _():
            cp = start_gather(t + 1, 0, sem0)
            compute(1)
            cp.wait()

        pltpu.sync_copy(out_v, out_hbm.at[pl.ds(base + t * CB, CB)])

    pl.loop(0, NCHUNK - 1)(chunk_body)

    @pl.when((NCHUNK - 1) % 2 == 0)
    def _():
        compute(0)

    @pl.when((NCHUNK - 1) % 2 == 1)
    def _():
        compute(1)

    pltpu.sync_copy(out_v, out_hbm.at[pl.ds(base + (NCHUNK - 1) * CB, CB)])


GT = 40


def _xpose_body(in_ref, out_ref):
    for g in range(GT):
        s = in_ref[pl.ds(g * POOL * POOL, POOL * POOL), :].astype(jnp.float32)
        s = jnp.concatenate([s, jnp.full((POOL, C), NEG, jnp.float32)], axis=0)
        out_ref[g] = jnp.transpose(s, (1, 0))[:, 0:POOL * POOL]


def _roi_corner_idx(rois):
    b = rois[:, 0].astype(jnp.int32)
    rs_w = jnp.round(rois[:, 1] * SCALE).astype(jnp.int32)
    rs_h = jnp.round(rois[:, 2] * SCALE).astype(jnp.int32)
    re_w = jnp.round(rois[:, 3] * SCALE).astype(jnp.int32)
    re_h = jnp.round(rois[:, 4] * SCALE).astype(jnp.int32)
    roi_w = jnp.maximum(re_w - rs_w + 1, 1).astype(jnp.float32)
    roi_h = jnp.maximum(re_h - rs_h + 1, 1).astype(jnp.float32)
    bin_w = roi_w / POOL
    bin_h = roi_h / POOL
    p = jnp.arange(POOL, dtype=jnp.float32)
    hstart = jnp.clip(jnp.floor(p[None, :] * bin_h[:, None]).astype(jnp.int32) + rs_h[:, None], 0, H)
    hend = jnp.clip(jnp.ceil((p[None, :] + 1.0) * bin_h[:, None]).astype(jnp.int32) + rs_h[:, None], 0, H)
    wstart = jnp.clip(jnp.floor(p[None, :] * bin_w[:, None]).astype(jnp.int32) + rs_w[:, None], 0, W)
    wend = jnp.clip(jnp.ceil((p[None, :] + 1.0) * bin_w[:, None]).astype(jnp.int32) + rs_w[:, None], 0, W)

    len_h = hend - hstart
    len_w = wend - wstart
    kh = (len_h >= 2).astype(jnp.int32) + (len_h >= 4).astype(jnp.int32)
    kw = (len_w >= 2).astype(jnp.int32) + (len_w >= 4).astype(jnp.int32)
    hA = jnp.clip(hstart, 0, H - 1)
    hB = jnp.clip(hend - (1 << kh), 0, H - 1)
    wA = jnp.clip(wstart, 0, W - 1)
    wB = jnp.clip(wend - (1 << kw), 0, W - 1)

    combo = (kh[:, :, None] * 3 + kw[:, None, :]) * B + b[:, None, None]
    cbase = combo * HW
    valid = (len_h > 0)[:, :, None] & (len_w > 0)[:, None, :]

    def corner(h, w):
        flat = cbase + (h[:, :, None] * W + w[:, None, :])
        return jnp.where(valid, flat, ZERO_ROW)

    c00 = corner(hA, wA)
    c01 = corner(hA, wB)
    c10 = corner(hB, wA)
    c11 = corner(hB, wB)
    idx = jnp.stack([c00, c01, c10, c11], axis=-1).reshape(NBINS * 4)
    pad = jnp.full((BINSP * 4 - NBINS * 4,), ZERO_ROW, jnp.int32)
    return jnp.concatenate([idx, pad])


def kernel(features, rois):
    fmap = jnp.transpose(features, (0, 2, 3, 1))
    idx = _roi_corner_idx(rois)

    table = pl.pallas_call(
        _table_body,
        in_specs=[pl.BlockSpec((B, H, W, C), lambda: (0, 0, 0, 0))],
        out_specs=pl.BlockSpec((NCOMBO * B + 1, H, W, C), lambda: (0, 0, 0, 0)),
        out_shape=jax.ShapeDtypeStruct((NCOMBO * B + 1, H, W, C), jnp.bfloat16),
    )(fmap)
    table_i32 = jax.lax.bitcast_convert_type(
        table.reshape(NCOMBO * B + 1, H, W, C // 2, 2), jnp.int32
    ).reshape((NCOMBO * B + 1) * HW, C // 2)

    mesh = plsc.VectorSubcoreMesh(core_axis_name="c", subcore_axis_name="s")
    cp = pltpu.CompilerParams()
    if "needs_layout_passes" in pltpu.CompilerParams.__dataclass_fields__:
        cp = dataclasses.replace(cp, needs_layout_passes=False)
    sc_gather = pl.kernel(
        _sc_body,
        out_type=jax.ShapeDtypeStruct((BINSP, C // 2), jnp.int32),
        mesh=mesh,
        compiler_params=cp,
        scratch_types=[
            pltpu.VMEM((2, CB * 4), jnp.int32),
            pltpu.VMEM((2, CB * 4, C // 2), jnp.int32),
            pltpu.VMEM((CB, C // 2), jnp.int32),
            pltpu.SemaphoreType.DMA,
            pltpu.SemaphoreType.DMA,
        ],
    )
    binmax_i32 = sc_gather(table_i32, idx)
    binmax = jax.lax.bitcast_convert_type(binmax_i32, jnp.bfloat16).reshape(BINSP, C)

    out = pl.pallas_call(
        _xpose_body,
        grid=(N // GT,),
        in_specs=[pl.BlockSpec((GT * POOL * POOL, C), lambda i: (i, 0))],
        out_specs=pl.BlockSpec((GT, C, POOL * POOL), lambda i: (i, 0, 0)),
        out_shape=jax.ShapeDtypeStruct((N, C, POOL * POOL), jnp.float32),
    )(binmax)
    return out.reshape(N, C, POOL, POOL)


# speedup vs baseline: 2.1841x; 2.1841x over previous
import dataclasses
import functools

import jax
import jax.numpy as jnp
from jax import lax
from jax.experimental import pallas as pl
from jax.experimental.pallas import tpu as pltpu
from jax.experimental.pallas import tpu_sc as plsc

POOL = 7
SCALE = 0.0625
B, C, H, W = 2, 256, 38, 38
N = 1000
NEG = jnp.finfo(jnp.float32).min

HW = H * W
NCOMBO = 9
ZERO_ROW = NCOMBO * B * HW

NBINS = N * POOL * POOL
NWORKERS = 32
CB = 32
BINS_PER_W = 1536
BINSP = NWORKERS * BINS_PER_W
NCHUNK = BINS_PER_W // CB


def _shift0(x, s, size):
    return jnp.concatenate([x[s:], jnp.broadcast_to(x[size - 1:], (s,) + x.shape[1:])], axis=0)


def _shift1(x, s, size):
    last = x[:, size - 1:]
    return jnp.concatenate([x[:, s:], jnp.broadcast_to(last, x.shape[:1] + (s,) + x.shape[2:])], axis=1)


def _pack_bf16_pair(x):
    u_lo = jax.lax.bitcast_convert_type(x[..., 0:C // 2], jnp.uint32)
    u_hi = jax.lax.bitcast_convert_type(x[..., C // 2:C], jnp.uint32)
    rne = lambda u: u + jnp.uint32(0x7FFF) + ((u >> 16) & jnp.uint32(1))
    lo16 = rne(u_lo) >> 16
    hi16 = rne(u_hi) & jnp.uint32(0xFFFF0000)
    return lo16 | hi16


def _table_body(fmap_ref, out_ref):
    for bb in range(B):
        f = fmap_ref[bb]
        th = f
        for kh in range(3):
            if kh:
                th = jnp.maximum(th, _shift0(th, 1 << (kh - 1), H))
            tw = th
            for kw in range(3):
                if kw:
                    tw = jnp.maximum(tw, _shift1(tw, 1 << (kw - 1), W))
                out_ref[(kh * 3 + kw) * B + bb] = _pack_bf16_pair(tw)
    out_ref[NCOMBO * B] = jnp.zeros((H, W, C // 2), jnp.uint32)


def _sc_body(table_hbm, idx_hbm, out_hbm, idx_v, rows_v, out_v, sem0, sem1):
    wid = lax.axis_index("s") * 2 + lax.axis_index("c")
    base = wid * BINS_PER_W

    def start_gather(t, buf, sem):
        pltpu.sync_copy(idx_hbm.at[pl.ds((base + t * CB) * 4, CB * 4)], idx_v.at[buf])
        return pltpu.async_copy(table_hbm.at[idx_v.at[buf]], rows_v.at[buf], sem)

    def compute(buf):
        def bin_body(i):
            for j in range(C // 32):
                sl = pl.ds(j * 16, 16)
                r = [plsc.bitcast(rows_v[buf, 4 * i + k, sl], jnp.bfloat16)
                     for k in range(4)]
                m = jnp.maximum(jnp.maximum(r[0], r[1]), jnp.maximum(r[2], r[3]))
                out_v[i, sl] = plsc.bitcast(m, jnp.int32)
        pl.loop(0, CB)(bin_body)

    start_gather(0, 0, sem0).wait()

    def chunk_body(t):
        @pl.when(t % 2 == 0)
        def _():
            cp = start_gather(t + 1, 1, sem1)
            compute(0)
            cp.wait()

        @pl.when(t % 2 == 1)
        def _():
            cp = start_gather(t + 1, 0, sem0)
            compute(1)
            cp.wait()

        pltpu.sync_copy(out_v, out_hbm.at[pl.ds(base + t * CB, CB)])

    pl.loop(0, NCHUNK - 1)(chunk_body)

    @pl.when((NCHUNK - 1) % 2 == 0)
    def _():
        compute(0)

    @pl.when((NCHUNK - 1) % 2 == 1)
    def _():
        compute(1)

    pltpu.sync_copy(out_v, out_hbm.at[pl.ds(base + (NCHUNK - 1) * CB, CB)])


GT = 40


def _xpose_body(in_ref, out_ref):
    for g in range(GT):
        x = in_ref[pl.ds(g * POOL * POOL, POOL * POOL), :]
        lo = jax.lax.bitcast_convert_type(x << 16, jnp.float32)
        hi = jax.lax.bitcast_convert_type(x & jnp.uint32(0xFFFF0000), jnp.float32)
        s = jnp.concatenate([lo, hi], axis=1)
        s = jnp.concatenate([s, jnp.full((POOL, C), NEG, jnp.float32)], axis=0)
        out_ref[g] = jnp.transpose(s, (1, 0))[:, 0:POOL * POOL]


def _roi_corner_idx(rois):
    b = rois[:, 0].astype(jnp.int32)
    rs_w = jnp.round(rois[:, 1] * SCALE).astype(jnp.int32)
    rs_h = jnp.round(rois[:, 2] * SCALE).astype(jnp.int32)
    re_w = jnp.round(rois[:, 3] * SCALE).astype(jnp.int32)
    re_h = jnp.round(rois[:, 4] * SCALE).astype(jnp.int32)
    roi_w = jnp.maximum(re_w - rs_w + 1, 1).astype(jnp.float32)
    roi_h = jnp.maximum(re_h - rs_h + 1, 1).astype(jnp.float32)
    bin_w = roi_w / POOL
    bin_h = roi_h / POOL
    p = jnp.arange(POOL, dtype=jnp.float32)
    hstart = jnp.clip(jnp.floor(p[None, :] * bin_h[:, None]).astype(jnp.int32) + rs_h[:, None], 0, H)
    hend = jnp.clip(jnp.ceil((p[None, :] + 1.0) * bin_h[:, None]).astype(jnp.int32) + rs_h[:, None], 0, H)
    wstart = jnp.clip(jnp.floor(p[None, :] * bin_w[:, None]).astype(jnp.int32) + rs_w[:, None], 0, W)
    wend = jnp.clip(jnp.ceil((p[None, :] + 1.0) * bin_w[:, None]).astype(jnp.int32) + rs_w[:, None], 0, W)

    len_h = hend - hstart
    len_w = wend - wstart
    kh = (len_h >= 2).astype(jnp.int32) + (len_h >= 4).astype(jnp.int32)
    kw = (len_w >= 2).astype(jnp.int32) + (len_w >= 4).astype(jnp.int32)
    hA = jnp.clip(hstart, 0, H - 1)
    hB = jnp.clip(hend - (1 << kh), 0, H - 1)
    wA = jnp.clip(wstart, 0, W - 1)
    wB = jnp.clip(wend - (1 << kw), 0, W - 1)

    combo = (kh[:, :, None] * 3 + kw[:, None, :]) * B + b[:, None, None]
    cbase = combo * HW
    valid = (len_h > 0)[:, :, None] & (len_w > 0)[:, None, :]

    def corner(h, w):
        flat = cbase + (h[:, :, None] * W + w[:, None, :])
        return jnp.where(valid, flat, ZERO_ROW)

    c00 = corner(hA, wA)
    c01 = corner(hA, wB)
    c10 = corner(hB, wA)
    c11 = corner(hB, wB)
    idx = jnp.stack([c00, c01, c10, c11], axis=-1).reshape(NBINS * 4)
    pad = jnp.full((BINSP * 4 - NBINS * 4,), ZERO_ROW, jnp.int32)
    return jnp.concatenate([idx, pad])


def kernel(features, rois):
    fmap = jnp.transpose(features, (0, 2, 3, 1))
    idx = _roi_corner_idx(rois)

    table_u32 = pl.pallas_call(
        _table_body,
        in_specs=[pl.BlockSpec((B, H, W, C), lambda: (0, 0, 0, 0))],
        out_specs=pl.BlockSpec((NCOMBO * B + 1, H, W, C // 2), lambda: (0, 0, 0, 0)),
        out_shape=jax.ShapeDtypeStruct((NCOMBO * B + 1, H, W, C // 2), jnp.uint32),
    )(fmap).reshape((NCOMBO * B + 1) * HW, C // 2)

    mesh = plsc.VectorSubcoreMesh(core_axis_name="c", subcore_axis_name="s")
    cp = pltpu.CompilerParams()
    if "needs_layout_passes" in pltpu.CompilerParams.__dataclass_fields__:
        cp = dataclasses.replace(cp, needs_layout_passes=False)
    sc_gather = pl.kernel(
        _sc_body,
        out_type=jax.ShapeDtypeStruct((BINSP, C // 2), jnp.uint32),
        mesh=mesh,
        compiler_params=cp,
        scratch_types=[
            pltpu.VMEM((2, CB * 4), jnp.int32),
            pltpu.VMEM((2, CB * 4, C // 2), jnp.uint32),
            pltpu.VMEM((CB, C // 2), jnp.uint32),
            pltpu.SemaphoreType.DMA,
            pltpu.SemaphoreType.DMA,
        ],
    )
    binmax = sc_gather(table_u32, idx)

    out = pl.pallas_call(
        _xpose_body,
        grid=(N // GT,),
        in_specs=[pl.BlockSpec((GT * POOL * POOL, C // 2), lambda i: (i, 0))],
        out_specs=pl.BlockSpec((GT, C, POOL * POOL), lambda i: (i, 0, 0)),
        out_shape=jax.ShapeDtypeStruct((N, C, POOL * POOL), jnp.float32),
    )(binmax)
    return out.reshape(N, C, POOL, POOL)


# speedup vs baseline: 3.6255x; 1.6599x over previous
import jax
import jax.numpy as jnp
from jax.experimental import pallas as pl
from jax.experimental.pallas import tpu as pltpu

POOL = 7
SCALE = 0.0625
B, C, H, W = 2, 256, 38, 38
N = 1000
G = 40
NEG = jnp.finfo(jnp.float32).min


def _roi_bins(rois):
    b = rois[:, 0].astype(jnp.int32)
    rs_w = jnp.round(rois[:, 1] * SCALE).astype(jnp.int32)
    rs_h = jnp.round(rois[:, 2] * SCALE).astype(jnp.int32)
    re_w = jnp.round(rois[:, 3] * SCALE).astype(jnp.int32)
    re_h = jnp.round(rois[:, 4] * SCALE).astype(jnp.int32)
    roi_w = jnp.maximum(re_w - rs_w + 1, 1).astype(jnp.float32)
    roi_h = jnp.maximum(re_h - rs_h + 1, 1).astype(jnp.float32)
    bin_w = roi_w / POOL
    bin_h = roi_h / POOL
    p = jnp.arange(POOL, dtype=jnp.float32)
    hstart = jnp.clip(jnp.floor(p[None, :] * bin_h[:, None]).astype(jnp.int32) + rs_h[:, None], 0, H)
    hend = jnp.clip(jnp.ceil((p[None, :] + 1.0) * bin_h[:, None]).astype(jnp.int32) + rs_h[:, None], 0, H)
    wstart = jnp.clip(jnp.floor(p[None, :] * bin_w[:, None]).astype(jnp.int32) + rs_w[:, None], 0, W)
    wend = jnp.clip(jnp.ceil((p[None, :] + 1.0) * bin_w[:, None]).astype(jnp.int32) + rs_w[:, None], 0, W)

    len_w = wend - wstart
    kw = (len_w >= 2).astype(jnp.int32) + (len_w >= 4).astype(jnp.int32)
    wA = jnp.clip(wstart, 0, W - 1)
    wB = jnp.clip(wend - (1 << kw), 0, W - 1)
    base = (kw * B + b[:, None]) * W
    rowA = base + wA
    rowB = base + wB

    hbase = (hstart // 8) * 8
    hlo = hstart - hbase
    hhi = jnp.minimum(hend - hbase, 16)

    valid = ((hend - hstart) > 0)[:, :, None] & (len_w > 0)[:, None, :]
    vbits = jnp.sum(valid.astype(jnp.int32) << jnp.arange(POOL)[None, None, :], axis=2)

    packed = jnp.concatenate([rowA, rowB, hbase, hlo, hhi, vbits], axis=1)
    return packed.reshape(N, 1, 6 * POOL).astype(jnp.int32)


def _kernel_body(idx_ref, fmap_ref, out_ref, tab_ref, colmax_ref):
    i = pl.program_id(0)

    @pl.when(i == 0)
    def _build_tables():
        for bb in range(B):
            f = fmap_ref[bb]
            t1 = jnp.maximum(f, jnp.concatenate([f[1:], f[W - 1:]], axis=0))
            t2 = jnp.maximum(t1, jnp.concatenate([t1[2:], t1[W - 2:]], axis=0))
            tab_ref[pl.ds((0 * B + bb) * W, W)] = f
            tab_ref[pl.ds((1 * B + bb) * W, W)] = t1
            tab_ref[pl.ds((2 * B + bb) * W, W)] = t2
        colmax_ref[:, :, 32:48, :] = jnp.full((2, POOL, 16, C), NEG, jnp.float32)

    iota16 = jax.lax.broadcasted_iota(jnp.int32, (1, 16, 1), 1)
    iota7 = jax.lax.broadcasted_iota(jnp.int32, (POOL, 1), 0)

    def one_roi(g, buf):
        for pw in range(POOL):
            rA = idx_ref[g, 0, pw]
            rB = idx_ref[g, 0, POOL + pw]
            colmax_ref[buf, pw, 0:H, :] = jnp.maximum(tab_ref[rA], tab_ref[rB])

        for ph in range(POOL):
            hb = pl.multiple_of(idx_ref[g, 0, 2 * POOL + ph], 8)
            lo = idx_ref[g, 0, 3 * POOL + ph]
            hi = idx_ref[g, 0, 4 * POOL + ph]
            vbits = idx_ref[g, 0, 5 * POOL + ph]
            win = colmax_ref[buf, :, pl.ds(hb, 16), :]
            madd = jnp.where((iota16 >= lo) & (iota16 < hi), 0.0, NEG)
            m = jnp.max(win + madd, axis=1)
            vmask = (jax.lax.shift_right_logical(vbits, iota7) & 1) > 0
            m = jnp.where(vmask, m, 0.0)
            out_ref[pl.ds(ph * POOL, POOL), g, :] = m

    def roi_pair(j, _):
        one_roi(2 * j, 0)
        one_roi(2 * j + 1, 1)
        return ()

    jax.lax.fori_loop(0, G // 2, roi_pair, (), unroll=False)


def kernel(features, rois):
    fmap = jnp.transpose(features, (0, 3, 2, 1))
    packed = _roi_bins(rois)

    out = pl.pallas_call(
        _kernel_body,
        grid=(N // G,),
        in_specs=[
            pl.BlockSpec((G, 1, 6 * POOL), lambda i: (i, 0, 0), memory_space=pltpu.SMEM),
            pl.BlockSpec((B, W, H, C), lambda i: (0, 0, 0, 0)),
        ],
        out_specs=pl.BlockSpec((POOL * POOL, G, C), lambda i: (0, i, 0)),
        out_shape=jax.ShapeDtypeStruct((POOL * POOL, N, C), jnp.float32),
        scratch_shapes=[
            pltpu.VMEM((3 * B * W, H, C), jnp.float32),
            pltpu.VMEM((2, POOL, 48, C), jnp.float32),
        ],
    )(packed, fmap)
    return jnp.transpose(out.reshape(POOL, POOL, N, C), (2, 3, 0, 1))
